# baseline (speedup 1.0000x reference)
import jax
import jax.numpy as jnp
from jax import lax
from jax.experimental import pallas as pl
from jax.experimental.pallas import tpu as pltpu
from jax.experimental.pallas import tpu_sc as plsc

_NC = 2
_NS = 16
_NW = _NC * _NS


def _sc_body(v_hbm, m_hbm, vo_hbm, mo_hbm, vbuf, mbuf, obuf, vsem):
    wid = lax.axis_index("s") * _NC + lax.axis_index("c")
    vrows = v_hbm.shape[0] // _NW
    melts = m_hbm.shape[0] // _NW
    vbase = wid * vrows
    mbase = wid * melts
    pltpu.async_copy(v_hbm.at[pl.ds(vbase, vrows)], vbuf, vsem).start()
    pltpu.sync_copy(m_hbm.at[pl.ds(mbase, melts)], mbuf)

    def _cast(i, carry):
        sl = pl.ds(i * 16, 16)
        obuf[sl] = jnp.sign(jnp.abs(mbuf[sl])).astype(jnp.int32)
        return carry

    lax.fori_loop(0, melts // 16, _cast, 0)
    pltpu.sync_copy(obuf, mo_hbm.at[pl.ds(mbase, melts)])
    pltpu.async_copy(v_hbm.at[pl.ds(vbase, vrows)], vbuf, vsem).wait()
    pltpu.sync_copy(vbuf, vo_hbm.at[pl.ds(vbase, vrows)])


def kernel(x, v, mask):
    B, C, P = x.shape
    Vc = v.shape[1]
    vrows = B * Vc
    melts = B * P
    v2 = v.reshape(vrows, P)
    m2 = mask.reshape(melts)
    sc = pl.kernel(
        _sc_body,
        out_type=[
            jax.ShapeDtypeStruct((vrows, P), v.dtype),
            jax.ShapeDtypeStruct((melts,), jnp.int32),
        ],
        mesh=plsc.VectorSubcoreMesh(core_axis_name="c", subcore_axis_name="s"),
        scratch_types=[
            pltpu.VMEM((vrows // _NW, P), v.dtype),
            pltpu.VMEM((melts // _NW,), jnp.float32),
            pltpu.VMEM((melts // _NW,), jnp.int32),
            pltpu.SemaphoreType.DMA,
        ],
    )
    vo, mo = sc(v2, m2)
    return (x, vo.reshape(B, Vc, P), mo.astype(jnp.bool_).reshape(B, 1, P))


# speedup vs baseline: 1.4857x; 1.4857x over previous
import jax
import jax.numpy as jnp
from jax.experimental import pallas as pl

_BB = 32


def _fused_kernel(x_ref, v_ref, m_ref, xo_ref, vo_ref, mo_ref):
    xo_ref[...] = x_ref[...]
    vo_ref[...] = v_ref[...]
    mo_ref[...] = m_ref[...] != 0.0


def kernel(x, v, mask):
    B, C, P = x.shape
    Vc = v.shape[1]
    m2 = mask.reshape(B, P)
    grid = (B // _BB,)
    xo, vo, mo = pl.pallas_call(
        _fused_kernel,
        grid=grid,
        in_specs=[
            pl.BlockSpec((_BB, C, P), lambda i: (i, 0, 0)),
            pl.BlockSpec((_BB, Vc, P), lambda i: (i, 0, 0)),
            pl.BlockSpec((_BB, P), lambda i: (i, 0)),
        ],
        out_specs=[
            pl.BlockSpec((_BB, C, P), lambda i: (i, 0, 0)),
            pl.BlockSpec((_BB, Vc, P), lambda i: (i, 0, 0)),
            pl.BlockSpec((_BB, P), lambda i: (i, 0)),
        ],
        out_shape=[
            jax.ShapeDtypeStruct((B, C, P), x.dtype),
            jax.ShapeDtypeStruct((B, Vc, P), v.dtype),
            jax.ShapeDtypeStruct((B, P), jnp.bool_),
        ],
    )(x, v, m2)
    return (xo, vo, mo.reshape(B, 1, P))
